# baseline (speedup 1.0000x reference)
import functools

import jax
import jax.numpy as jnp
from jax import lax
from jax.experimental import pallas as pl
from jax.experimental.pallas import tpu as pltpu
from jax.experimental.pallas import tpu_sc as plsc

DIM = 768
W2 = DIM // 2
NC = 2
NS = 16
NW = NC * NS
CHUNK = 40
NBUF = 2
CB = 194
PCH = 118


@functools.lru_cache(maxsize=None)
def _make_pack(v):
    blocks_total = v * DIM // 32
    bpw = -(-blocks_total // NW)
    assert bpw == CB * PCH

    @functools.partial(
        pl.kernel,
        mesh=plsc.VectorSubcoreMesh(core_axis_name="c", subcore_axis_name="s"),
        out_type=jax.ShapeDtypeStruct((v * W2,), jnp.int32),
        scratch_types=[pltpu.VMEM((CB * 32,), jnp.float32) for _ in range(NBUF)]
        + [pltpu.VMEM((CB * 16,), jnp.int32) for _ in range(NBUF)]
        + [pltpu.SemaphoreType.DMA for _ in range(2 * NBUF)],
    )
    def pack_kernel(table_hbm, packed_hbm, *scratch):
        fbufs = scratch[:NBUF]
        obufs = scratch[NBUF:2 * NBUF]
        fsem = scratch[2 * NBUF:3 * NBUF]
        osem = scratch[3 * NBUF:]
        wid = lax.axis_index("s") * NC + lax.axis_index("c")
        start = jnp.minimum(wid * bpw, blocks_total - bpw)

        def fire_in(c, b):
            pltpu.async_copy(
                table_hbm.at[pl.ds((start + c * CB) * 32, CB * 32)],
                fbufs[b], fsem[b])

        def wait_in(b):
            pltpu.make_async_copy(table_hbm.at[pl.ds(0, CB * 32)], fbufs[b],
                                  fsem[b]).wait()

        def fire_out(c, b):
            pltpu.async_copy(
                obufs[b],
                packed_hbm.at[pl.ds((start + c * CB) * 16, CB * 16)],
                osem[b])

        def wait_out(b):
            pltpu.make_async_copy(obufs[b], packed_hbm.at[pl.ds(0, CB * 16)],
                                  osem[b]).wait()

        def to_bf16_bits(x):
            u = jax.lax.bitcast_convert_type(x, jnp.int32)
            return u + jnp.int32(0x7FFF) + ((u >> 16) & jnp.int32(1))

        def pack_chunk(b):
            fb = fbufs[b]
            ob = obufs[b]

            @plsc.parallel_loop(0, CB, step=1, unroll=4)
            def blk_body(j):
                e0 = fb[pl.ds(j * 32, 16)]
                e1 = fb[pl.ds(j * 32 + 16, 16)]
                a0 = (to_bf16_bits(e0) >> 16) & jnp.int32(0xFFFF)
                a1 = to_bf16_bits(e1) & jnp.int32(-65536)
                ob[pl.ds(j * 16, 16)] = a0 | a1

        for b in range(NBUF):
            fire_in(b, b)

        def body(s, carry):
            for b in range(NBUF):
                c = s * NBUF + b
                wait_in(b)

                @pl.when(c >= NBUF)
                def _():
                    wait_out(b)

                pack_chunk(b)
                fire_out(c, b)

                @pl.when(c + NBUF < PCH)
                def _():
                    fire_in(c + NBUF, b)

            return carry

        lax.fori_loop(0, PCH // NBUF, body, 0)

        for b in range(NBUF):
            wait_out(b)

    return pack_kernel


@functools.lru_cache(maxsize=None)
def _make_gather(n_rows, v):
    b_per_w = n_rows // NW
    n_chunks = b_per_w // CHUNK
    n_super = n_chunks // NBUF

    @functools.partial(
        pl.kernel,
        mesh=plsc.VectorSubcoreMesh(core_axis_name="c", subcore_axis_name="s"),
        out_type=jax.ShapeDtypeStruct((n_rows * DIM,), jnp.int32),
        scratch_types=[
            pltpu.VMEM((b_per_w,), jnp.int32),
        ]
        + [pltpu.VMEM((CHUNK, W2), jnp.int32) for _ in range(NBUF)]
        + [pltpu.VMEM((CHUNK * DIM,), jnp.int32) for _ in range(NBUF)]
        + [pltpu.SemaphoreType.DMA for _ in range(2 * NBUF)],
    )
    def gather_kernel(idx_hbm, table_hbm, out_hbm, idx_v, *scratch):
        gbufs = scratch[:NBUF]
        sbufs = scratch[NBUF:2 * NBUF]
        gsem = scratch[2 * NBUF:3 * NBUF]
        ssem = scratch[3 * NBUF:]
        wid = lax.axis_index("s") * NC + lax.axis_index("c")
        base = wid * b_per_w
        pltpu.sync_copy(idx_hbm.at[pl.ds(base, b_per_w)], idx_v)

        def fire_gather(g, b):
            pltpu.async_copy(
                table_hbm.at[idx_v.at[pl.ds(g * CHUNK, CHUNK)]],
                gbufs[b], gsem[b],
            )

        def wait_gather(b):
            pltpu.make_async_copy(
                table_hbm.at[idx_v.at[pl.ds(0, CHUNK)]], gbufs[b], gsem[b]
            ).wait()

        def fire_store(g, b):
            pltpu.async_copy(
                sbufs[b],
                out_hbm.at[pl.ds((base + g * CHUNK) * DIM, CHUNK * DIM)],
                ssem[b],
            )

        def wait_store(b):
            pltpu.make_async_copy(
                sbufs[b], out_hbm.at[pl.ds(0, CHUNK * DIM)], ssem[b]
            ).wait()

        def convert(b):
            gb = gbufs[b]
            sb = sbufs[b]

            @plsc.parallel_loop(0, CHUNK, step=1, unroll=4)
            def row_body(r):
                for k in range(W2 // 16):
                    u = gb[r, pl.ds(k * 16, 16)]
                    o = r * DIM + k * 32
                    sb[pl.ds(o, 16)] = u << 16
                    sb[pl.ds(o + 16, 16)] = u & jnp.int32(-65536)

        for b in range(NBUF):
            fire_gather(b, b)

        def body(s, carry):
            for b in range(NBUF):
                g = s * NBUF + b
                wait_gather(b)

                @pl.when(g >= NBUF)
                def _():
                    wait_store(b)

                convert(b)
                fire_store(g, b)

                @pl.when(g + NBUF < n_chunks)
                def _():
                    fire_gather(g + NBUF, b)

            return carry

        lax.fori_loop(0, n_super, body, 0)

        for b in range(NBUF):
            wait_store(b)

    return gather_kernel


def kernel(token_ids, table):
    b, l = token_ids.shape
    n_rows = b * l
    v = table.shape[0]
    idx = token_ids.reshape(-1).astype(jnp.int32)
    packed = _make_pack(v)(table.reshape(-1)).reshape(v, W2)
    out = _make_gather(n_rows, v)(idx, packed)
    return jax.lax.bitcast_convert_type(out, jnp.float32).reshape(b, l, DIM)


# speedup vs baseline: 2.6844x; 2.6844x over previous
import functools

import jax
import jax.numpy as jnp
from jax import lax
from jax.experimental import pallas as pl
from jax.experimental.pallas import tpu as pltpu
from jax.experimental.pallas import tpu_sc as plsc

DIM = 768
W2 = DIM // 2
NC = 2
NS = 16
NW = NC * NS
CHUNK = 40
NBUF = 2
CB = 194
PCH = 118


@functools.lru_cache(maxsize=None)
def _make_pack(v):
    blocks_total = v * DIM // 32
    bpw = -(-blocks_total // NW)
    assert bpw == CB * PCH

    @functools.partial(
        pl.kernel,
        mesh=plsc.VectorSubcoreMesh(core_axis_name="c", subcore_axis_name="s"),
        out_type=jax.ShapeDtypeStruct((v * W2,), jnp.int32),
        scratch_types=[pltpu.VMEM((CB * 32,), jnp.float32) for _ in range(NBUF)]
        + [pltpu.VMEM((CB * 16,), jnp.int32) for _ in range(NBUF)]
        + [pltpu.SemaphoreType.DMA for _ in range(2 * NBUF)],
    )
    def pack_kernel(table_hbm, packed_hbm, *scratch):
        fbufs = scratch[:NBUF]
        obufs = scratch[NBUF:2 * NBUF]
        fsem = scratch[2 * NBUF:3 * NBUF]
        osem = scratch[3 * NBUF:]
        wid = lax.axis_index("s") * NC + lax.axis_index("c")
        start = jnp.minimum(wid * bpw, blocks_total - bpw)

        def fire_in(c, b):
            pltpu.async_copy(
                table_hbm.at[pl.ds((start + c * CB) * 32, CB * 32)],
                fbufs[b], fsem[b])

        def wait_in(b):
            pltpu.make_async_copy(table_hbm.at[pl.ds(0, CB * 32)], fbufs[b],
                                  fsem[b]).wait()

        def fire_out(c, b):
            pltpu.async_copy(
                obufs[b],
                packed_hbm.at[pl.ds((start + c * CB) * 16, CB * 16)],
                osem[b])

        def wait_out(b):
            pltpu.make_async_copy(obufs[b], packed_hbm.at[pl.ds(0, CB * 16)],
                                  osem[b]).wait()

        def to_bf16_bits(x):
            u = jax.lax.bitcast_convert_type(x, jnp.int32)
            return u + jnp.int32(0x7FFF) + ((u >> 16) & jnp.int32(1))

        def pack_chunk(b):
            fb = fbufs[b]
            ob = obufs[b]

            @plsc.parallel_loop(0, CB, step=1, unroll=4)
            def blk_body(j):
                e0 = fb[pl.ds(j * 32, 16)]
                e1 = fb[pl.ds(j * 32 + 16, 16)]
                a0 = (to_bf16_bits(e0) >> 16) & jnp.int32(0xFFFF)
                a1 = to_bf16_bits(e1) & jnp.int32(-65536)
                ob[pl.ds(j * 16, 16)] = a0 | a1

        for b in range(NBUF):
            fire_in(b, b)

        def body(s, carry):
            for b in range(NBUF):
                c = s * NBUF + b
                wait_in(b)

                @pl.when(c >= NBUF)
                def _():
                    wait_out(b)

                pack_chunk(b)
                fire_out(c, b)

                @pl.when(c + NBUF < PCH)
                def _():
                    fire_in(c + NBUF, b)

            return carry

        lax.fori_loop(0, PCH // NBUF, body, 0)

        for b in range(NBUF):
            wait_out(b)

    return pack_kernel


@functools.lru_cache(maxsize=None)
def _make_gather(n_rows, v):
    b_per_w = n_rows // NW
    n_chunks = b_per_w // CHUNK
    n_super = n_chunks // NBUF

    @functools.partial(
        pl.kernel,
        mesh=plsc.VectorSubcoreMesh(core_axis_name="c", subcore_axis_name="s"),
        out_type=jax.ShapeDtypeStruct((n_rows, DIM), jnp.float32),
        scratch_types=[
            pltpu.VMEM((b_per_w,), jnp.int32),
        ]
        + [pltpu.VMEM((CHUNK, W2), jnp.int32) for _ in range(NBUF)]
        + [pltpu.VMEM((CHUNK, DIM), jnp.float32) for _ in range(NBUF)]
        + [pltpu.SemaphoreType.DMA for _ in range(2 * NBUF)],
    )
    def gather_kernel(idx_hbm, table_hbm, out_hbm, idx_v, *scratch):
        gbufs = scratch[:NBUF]
        sbufs = scratch[NBUF:2 * NBUF]
        gsem = scratch[2 * NBUF:3 * NBUF]
        ssem = scratch[3 * NBUF:]
        wid = lax.axis_index("s") * NC + lax.axis_index("c")
        base = wid * b_per_w
        pltpu.sync_copy(idx_hbm.at[pl.ds(base, b_per_w)], idx_v)

        def fire_gather(g, b):
            pltpu.async_copy(
                table_hbm.at[idx_v.at[pl.ds(g * CHUNK, CHUNK)]],
                gbufs[b], gsem[b],
            )

        def wait_gather(b):
            pltpu.make_async_copy(
                table_hbm.at[idx_v.at[pl.ds(0, CHUNK)]], gbufs[b], gsem[b]
            ).wait()

        def fire_store(g, b):
            pltpu.async_copy(
                sbufs[b],
                out_hbm.at[pl.ds(base + g * CHUNK, CHUNK)],
                ssem[b],
            )

        def wait_store(b):
            pltpu.make_async_copy(
                sbufs[b], out_hbm.at[pl.ds(0, CHUNK)], ssem[b]
            ).wait()

        def convert(b):
            gb = gbufs[b]
            sb = sbufs[b]

            @plsc.parallel_loop(0, CHUNK, step=1, unroll=4)
            def row_body(r):
                for k in range(W2 // 16):
                    u = gb[r, pl.ds(k * 16, 16)]
                    lo = jax.lax.bitcast_convert_type(u << 16, jnp.float32)
                    hi = jax.lax.bitcast_convert_type(
                        u & jnp.int32(-65536), jnp.float32)
                    sb[r, pl.ds(k * 32, 16)] = lo
                    sb[r, pl.ds(k * 32 + 16, 16)] = hi

        for b in range(NBUF):
            fire_gather(b, b)

        def body(s, carry):
            for b in range(NBUF):
                g = s * NBUF + b
                wait_gather(b)

                @pl.when(g >= NBUF)
                def _():
                    wait_store(b)

                convert(b)
                fire_store(g, b)

                @pl.when(g + NBUF < n_chunks)
                def _():
                    fire_gather(g + NBUF, b)

            return carry

        lax.fori_loop(0, n_super, body, 0)

        for b in range(NBUF):
            wait_store(b)

    return gather_kernel


def kernel(token_ids, table):
    b, l = token_ids.shape
    n_rows = b * l
    v = table.shape[0]
    idx = token_ids.reshape(-1).astype(jnp.int32)
    packed = _make_pack(v)(table.reshape(-1)).reshape(v, W2)
    out = _make_gather(n_rows, v)(idx, packed)
    return out.reshape(b, l, DIM)


# speedup vs baseline: 3.0358x; 1.1309x over previous
import functools

import jax
import jax.numpy as jnp
from jax import lax
from jax.experimental import pallas as pl
from jax.experimental.pallas import tpu as pltpu
from jax.experimental.pallas import tpu_sc as plsc

DIM = 768
W2 = DIM // 2
NC = 2
NS = 16
NW = NC * NS
CHUNK = 40
NBUF = 2
PRW = 48
PCH = 20
RW = PRW * PCH
PADV = NW * RW


@functools.lru_cache(maxsize=None)
def _make_pack(v):
    t_norm = (v // PRW) * PRW
    sh_out = t_norm + 8
    in_tail = v - PRW

    @functools.partial(
        pl.kernel,
        mesh=plsc.VectorSubcoreMesh(core_axis_name="c", subcore_axis_name="s"),
        out_type=jax.ShapeDtypeStruct((PADV, W2), jnp.int32),
        scratch_types=[pltpu.VMEM((PRW * DIM,), jnp.float32)
                       for _ in range(NBUF)]
        + [pltpu.VMEM((PRW, W2), jnp.int32) for _ in range(NBUF)]
        + [pltpu.SemaphoreType.DMA for _ in range(2 * NBUF)],
    )
    def pack_kernel(table_hbm, packed_hbm, *scratch):
        fbufs = scratch[:NBUF]
        obufs = scratch[NBUF:2 * NBUF]
        fsem = scratch[2 * NBUF:3 * NBUF]
        osem = scratch[3 * NBUF:]
        wid = lax.axis_index("s") * NC + lax.axis_index("c")
        def chunk_params(c):
            out_row = wid * RW + c * PRW
            shift = out_row > in_tail
            in_row = jnp.minimum(out_row, in_tail)
            out_row8 = jnp.where(shift, sh_out // 8,
                                 wid * (RW // 8) + c * (PRW // 8)) * 8
            live = out_row <= t_norm
            return in_row, out_row8, live

        def fire_in(c, b):
            in_row, _, live = chunk_params(c)

            @pl.when(live)
            def _():
                pltpu.async_copy(
                    table_hbm.at[pl.ds(in_row * DIM, PRW * DIM)],
                    fbufs[b], fsem[b])

        def wait_in(c, b):
            _, _, live = chunk_params(c)

            @pl.when(live)
            def _():
                pltpu.make_async_copy(
                    table_hbm.at[pl.ds(0, PRW * DIM)], fbufs[b], fsem[b]
                ).wait()

        def fire_out(c, b):
            _, out_row8, live = chunk_params(c)

            @pl.when(live)
            def _():
                pltpu.async_copy(
                    obufs[b], packed_hbm.at[pl.ds(out_row8, PRW)], osem[b])

        def wait_out(c, b):
            _, _, live = chunk_params(c)

            @pl.when(live)
            def _():
                pltpu.make_async_copy(
                    obufs[b], packed_hbm.at[pl.ds(0, PRW)], osem[b]
                ).wait()

        def to_bf16_bits(x):
            u = jax.lax.bitcast_convert_type(x, jnp.int32)
            return u + jnp.int32(0x7FFF) + ((u >> 16) & jnp.int32(1))

        def pack_chunk(c, b):
            _, _, live = chunk_params(c)
            fb = fbufs[b]
            ob = obufs[b]

            @pl.when(live)
            def _():
                @plsc.parallel_loop(0, PRW, step=1, unroll=2)
                def row_body(r):
                    for k in range(W2 // 16):
                        e0 = fb[pl.ds(r * DIM + k * 32, 16)]
                        e1 = fb[pl.ds(r * DIM + k * 32 + 16, 16)]
                        a0 = (to_bf16_bits(e0) >> 16) & jnp.int32(0xFFFF)
                        a1 = to_bf16_bits(e1) & jnp.int32(-65536)
                        ob[r, pl.ds(k * 16, 16)] = a0 | a1

        for b in range(NBUF):
            fire_in(b, b)

        def body(s, carry):
            for b in range(NBUF):
                c = s * NBUF + b
                wait_in(c, b)

                @pl.when(c >= NBUF)
                def _():
                    wait_out(c - NBUF, b)

                pack_chunk(c, b)
                fire_out(c, b)

                @pl.when(c + NBUF < PCH)
                def _():
                    fire_in(c + NBUF, b)

            return carry

        lax.fori_loop(0, PCH // NBUF, body, 0)

        for b in range(NBUF):
            wait_out(PCH - NBUF + b, b)

    return pack_kernel


@functools.lru_cache(maxsize=None)
def _make_gather(n_rows, v):
    b_per_w = n_rows // NW
    n_chunks = b_per_w // CHUNK
    n_super = n_chunks // NBUF

    @functools.partial(
        pl.kernel,
        mesh=plsc.VectorSubcoreMesh(core_axis_name="c", subcore_axis_name="s"),
        out_type=jax.ShapeDtypeStruct((n_rows, DIM), jnp.float32),
        scratch_types=[
            pltpu.VMEM((b_per_w,), jnp.int32),
        ]
        + [pltpu.VMEM((CHUNK, W2), jnp.int32) for _ in range(NBUF)]
        + [pltpu.VMEM((CHUNK, DIM), jnp.float32) for _ in range(NBUF)]
        + [pltpu.SemaphoreType.DMA for _ in range(2 * NBUF)],
    )
    def gather_kernel(idx_hbm, table_hbm, out_hbm, idx_v, *scratch):
        gbufs = scratch[:NBUF]
        sbufs = scratch[NBUF:2 * NBUF]
        gsem = scratch[2 * NBUF:3 * NBUF]
        ssem = scratch[3 * NBUF:]
        wid = lax.axis_index("s") * NC + lax.axis_index("c")
        base = wid * b_per_w
        pltpu.sync_copy(idx_hbm.at[pl.ds(base, b_per_w)], idx_v)

        t_norm = (v // PRW) * PRW
        delta = jnp.int32(t_norm + 8 - (v - PRW))

        @plsc.parallel_loop(0, b_per_w // 16, step=1, unroll=8)
        def remap_body(i):
            u = idx_v[pl.ds(i * 16, 16)]
            idx_v[pl.ds(i * 16, 16)] = u + jnp.where(
                u >= jnp.int32(t_norm), delta, jnp.int32(0))

        def fire_gather(g, b):
            pltpu.async_copy(
                table_hbm.at[idx_v.at[pl.ds(g * CHUNK, CHUNK)]],
                gbufs[b], gsem[b],
            )

        def wait_gather(b):
            pltpu.make_async_copy(
                table_hbm.at[idx_v.at[pl.ds(0, CHUNK)]], gbufs[b], gsem[b]
            ).wait()

        def fire_store(g, b):
            pltpu.async_copy(
                sbufs[b],
                out_hbm.at[pl.ds(base + g * CHUNK, CHUNK)],
                ssem[b],
            )

        def wait_store(b):
            pltpu.make_async_copy(
                sbufs[b], out_hbm.at[pl.ds(0, CHUNK)], ssem[b]
            ).wait()

        def convert(b):
            gb = gbufs[b]
            sb = sbufs[b]

            @plsc.parallel_loop(0, CHUNK, step=1, unroll=4)
            def row_body(r):
                for k in range(W2 // 16):
                    u = gb[r, pl.ds(k * 16, 16)]
                    lo = jax.lax.bitcast_convert_type(u << 16, jnp.float32)
                    hi = jax.lax.bitcast_convert_type(
                        u & jnp.int32(-65536), jnp.float32)
                    sb[r, pl.ds(k * 32, 16)] = lo
                    sb[r, pl.ds(k * 32 + 16, 16)] = hi

        for b in range(NBUF):
            fire_gather(b, b)

        def body(s, carry):
            for b in range(NBUF):
                g = s * NBUF + b
                wait_gather(b)

                @pl.when(g >= NBUF)
                def _():
                    wait_store(b)

                convert(b)
                fire_store(g, b)

                @pl.when(g + NBUF < n_chunks)
                def _():
                    fire_gather(g + NBUF, b)

            return carry

        lax.fori_loop(0, n_super, body, 0)

        for b in range(NBUF):
            wait_store(b)

    return gather_kernel


def kernel(token_ids, table):
    b, l = token_ids.shape
    n_rows = b * l
    v = table.shape[0]
    idx = token_ids.reshape(-1).astype(jnp.int32)
    packed = _make_pack(v)(table.reshape(-1))
    out = _make_gather(n_rows, v)(idx, packed)
    return out.reshape(b, l, DIM)


# speedup vs baseline: 3.5340x; 1.1641x over previous
import functools

import jax
import jax.numpy as jnp
from jax import lax
from jax.experimental import pallas as pl
from jax.experimental.pallas import tpu as pltpu
from jax.experimental.pallas import tpu_sc as plsc

DIM = 768
NC = 2
NS = 16
NW = NC * NS
CHUNK = 40
NBUF = 4
LAG = 2


@functools.lru_cache(maxsize=None)
def _make_gather(n_rows):
    b_per_w = n_rows // NW
    n_chunks = b_per_w // CHUNK
    n_super = n_chunks // NBUF
    mesh = plsc.VectorSubcoreMesh(core_axis_name="c", subcore_axis_name="s")

    @functools.partial(
        pl.kernel,
        mesh=mesh,
        out_type=jax.ShapeDtypeStruct((n_rows, DIM), jnp.float32),
        scratch_types=[
            pltpu.VMEM((b_per_w,), jnp.int32),
        ]
        + [pltpu.VMEM((CHUNK, DIM), jnp.float32) for _ in range(NBUF)]
        + [pltpu.SemaphoreType.DMA for _ in range(2 * NBUF)],
    )
    def gather_kernel(idx_hbm, table_hbm, out_hbm, idx_v, *scratch):
        bufs = scratch[:NBUF]
        gsem = scratch[NBUF:2 * NBUF]
        ssem = scratch[2 * NBUF:]
        wid = lax.axis_index("s") * NC + lax.axis_index("c")
        base = wid * b_per_w
        pltpu.sync_copy(idx_hbm.at[pl.ds(base, b_per_w)], idx_v)

        def fire_gather(g, b):
            pltpu.async_copy(
                table_hbm.at[idx_v.at[pl.ds(g * CHUNK, CHUNK)]], bufs[b],
                gsem[b])

        def fire_store(g, b):
            pltpu.async_copy(bufs[b], out_hbm.at[pl.ds(base + g * CHUNK, CHUNK)],
                             ssem[b])

        def wait_gather(b):
            pltpu.make_async_copy(
                table_hbm.at[idx_v.at[pl.ds(0, CHUNK)]], bufs[b], gsem[b]
            ).wait()

        def wait_store(b):
            pltpu.make_async_copy(
                bufs[b], out_hbm.at[pl.ds(base, CHUNK)], ssem[b]
            ).wait()

        def body(s, carry):
            for b in range(NBUF):
                g = s * NBUF + b

                @pl.when(g >= NBUF)
                def _():
                    wait_store(b)

                fire_gather(g, b)
                bs = (b - LAG) % NBUF

                @pl.when(g >= LAG)
                def _():
                    wait_gather(bs)
                    fire_store(g - LAG, bs)

            return carry

        lax.fori_loop(0, n_super, body, 0)

        for j in range(LAG):
            g = n_chunks - LAG + j
            b = g % NBUF
            wait_gather(b)
            fire_store(g, b)
        for b in range(NBUF):
            wait_store(b)

    return gather_kernel


def kernel(token_ids, table):
    b, l = token_ids.shape
    idx = token_ids.reshape(-1).astype(jnp.int32)
    out = _make_gather(b * l)(idx, table)
    return out.reshape(b, l, DIM)
